# baseline (speedup 1.0000x reference)
import jax
import jax.numpy as jnp
from jax import lax
from jax.experimental import pallas as pl
from jax.experimental.pallas import tpu as pltpu
from jax.experimental.pallas import tpu_sc as plsc

D_VOC = 131
D_MOD = 32
N_ROWS = 16384
ROW_LEN = 200
_STRIDE = 33

_info = plsc.get_sparse_core_info()
_NC, _NS = _info.num_cores, _info.num_subcores
_NW = _NC * _NS

_I_PER_W = N_ROWS // _NW
_IB = 128
_JB = 8
_N_IBLK = _I_PER_W // _IB
_N_JBLK = ROW_LEN // _JB


_N_CHUNKS = _N_IBLK * _N_JBLK


def _emb_body(tokens_hbm, table_hbm, out_hbm, table_v,
              tok_v0, tok_v1, buf_v0, buf_v1, sem0, sem1, tsem0, tsem1):
    wid = lax.axis_index("s") * _NC + lax.axis_index("c")
    i_base = wid * _I_PER_W

    pltpu.sync_copy(table_hbm, table_v)
    toks = [tok_v0, tok_v1]
    bufs = [buf_v0, buf_v1]
    sems = [sem0, sem1]
    tsems = [tsem0, tsem1]

    def tok_slice(q):
        blk = q // _N_JBLK
        jq = q % _N_JBLK
        return tokens_hbm.at[pl.ds(jq * _JB, _JB),
                             pl.ds(i_base + blk * _IB, _IB)]

    def prefetch(q, b):
        pltpu.async_copy(tok_slice(q), toks[b], tsems[b])

    def compute(q, b):
        blk = q // _N_JBLK
        jq = q % _N_JBLK
        i0 = i_base + blk * _IB
        j0 = jq * _JB
        tok_v, buf_v = toks[b], bufs[b]
        pltpu.make_async_copy(tok_slice(q), tok_v, tsems[b]).wait()

        @plsc.parallel_loop(0, _JB, unroll=4)
        def jcol(jl):
            for g in range(_IB // 16):
                tok16 = tok_v[jl, pl.ds(g * 16, 16)]
                tok33 = tok16 * _STRIDE
                vals = [plsc.load_gather(table_v, [tok33 + d])
                        for d in range(D_MOD)]
                for d in range(D_MOD):
                    buf_v[jl, d, pl.ds(g * 16, 16)] = vals[d]
        pltpu.async_copy(
            buf_v, out_hbm.at[pl.ds(j0, _JB), :, pl.ds(i0, _IB)], sems[b])

        @pl.when(q + 2 < _N_CHUNKS)
        def _():
            prefetch(q + 2, b)

    def drain(b):
        pltpu.make_async_copy(
            bufs[b], out_hbm.at[pl.ds(0, _JB), :, pl.ds(i_base, _IB)],
            sems[b]).wait()

    prefetch(0, 0)
    prefetch(1, 1)

    def super_step(s, carry):
        for b in range(2):
            q = s * 2 + b

            @pl.when(s > 0)
            def _():
                drain(b)
            compute(q, b)
        return carry

    lax.fori_loop(0, _N_CHUNKS // 2, super_step, 0)
    drain(0)
    drain(1)


@jax.jit
def _emb_call(tokens_t, table_pad):
    mesh = plsc.VectorSubcoreMesh(core_axis_name="c", subcore_axis_name="s")
    f = pl.kernel(
        _emb_body,
        out_type=jax.ShapeDtypeStruct((ROW_LEN, D_MOD, N_ROWS), jnp.float32),
        mesh=mesh,
        scratch_types=[
            pltpu.VMEM((D_VOC * _STRIDE + 5,), jnp.float32),
            pltpu.VMEM((_JB, _IB), jnp.int32),
            pltpu.VMEM((_JB, _IB), jnp.int32),
            pltpu.VMEM((_JB, D_MOD, _IB), jnp.float32),
            pltpu.VMEM((_JB, D_MOD, _IB), jnp.float32),
            pltpu.SemaphoreType.DMA,
            pltpu.SemaphoreType.DMA,
            pltpu.SemaphoreType.DMA,
            pltpu.SemaphoreType.DMA,
        ],
        compiler_params=pltpu.CompilerParams(needs_layout_passes=False),
    )
    return f(tokens_t, table_pad)


def kernel(tokens, weight):
    tokens_t = tokens.T.astype(jnp.int32)
    table_pad = jnp.pad(weight, ((0, 0), (0, 1))).reshape(-1)
    table_pad = jnp.pad(table_pad, (0, 5))
    out_phys = _emb_call(tokens_t, table_pad)
    return jnp.transpose(out_phys, (2, 0, 1))


# speedup vs baseline: 2.3775x; 2.3775x over previous
import jax
import jax.numpy as jnp
from jax import lax
from jax.experimental import pallas as pl
from jax.experimental.pallas import tpu as pltpu
from jax.experimental.pallas import tpu_sc as plsc

D_VOC = 131
D_MOD = 32
N_ROWS = 16384
ROW_LEN = 200
_STRIDE = 33

_info = plsc.get_sparse_core_info()
_NC, _NS = _info.num_cores, _info.num_subcores
_NW = _NC * _NS

_I_PER_W = N_ROWS // _NW
_IB = 128
_JB = 8
_N_IBLK = _I_PER_W // _IB
_N_JBLK = ROW_LEN // _JB


_N_CHUNKS = _N_IBLK * _N_JBLK


def _emb_body(tokens_hbm, table_hbm, out_hbm, table_v,
              tok_v0, tok_v1, buf_v0, buf_v1, sem0, sem1, tsem0, tsem1):
    wid = lax.axis_index("s") * _NC + lax.axis_index("c")
    i_base = wid * _I_PER_W

    pltpu.sync_copy(table_hbm, table_v)
    toks = [tok_v0, tok_v1]
    bufs = [buf_v0, buf_v1]
    sems = [sem0, sem1]
    tsems = [tsem0, tsem1]

    def tok_slice(q):
        blk = q // _N_JBLK
        jq = q % _N_JBLK
        return tokens_hbm.at[pl.ds(jq * _JB, _JB),
                             pl.ds(i_base + blk * _IB, _IB)]

    def prefetch(q, b):
        pltpu.async_copy(tok_slice(q), toks[b], tsems[b])

    def compute(q, b):
        blk = q // _N_JBLK
        jq = q % _N_JBLK
        i0 = i_base + blk * _IB
        j0 = jq * _JB
        tok_v, buf_v = toks[b], bufs[b]
        pltpu.make_async_copy(tok_slice(q), tok_v, tsems[b]).wait()

        @plsc.parallel_loop(0, _JB, unroll=2)
        def jcol(jl):
            for g in range(_IB // 16):
                tok16 = tok_v[jl, pl.ds(g * 16, 16)]
                tok33 = tok16 * _STRIDE
                for dh in range(0, D_MOD, 16):
                    vals = [plsc.load_gather(table_v, [tok33 + (dh + d)])
                            for d in range(16)]
                    for d in range(16):
                        buf_v[jl, dh + d, pl.ds(g * 16, 16)] = vals[d]
        pltpu.async_copy(
            buf_v, out_hbm.at[pl.ds(j0, _JB), :, pl.ds(i0, _IB)], sems[b])

        @pl.when(q + 2 < _N_CHUNKS)
        def _():
            prefetch(q + 2, b)

    def drain(b):
        pltpu.make_async_copy(
            bufs[b], out_hbm.at[pl.ds(0, _JB), :, pl.ds(i_base, _IB)],
            sems[b]).wait()

    prefetch(0, 0)
    prefetch(1, 1)

    def super_step(s, carry):
        for b in range(2):
            q = s * 2 + b

            @pl.when(s > 0)
            def _():
                drain(b)
            compute(q, b)
        return carry

    lax.fori_loop(0, _N_CHUNKS // 2, super_step, 0)
    drain(0)
    drain(1)


@jax.jit
def _emb_call(tokens_t, table_pad):
    mesh = plsc.VectorSubcoreMesh(core_axis_name="c", subcore_axis_name="s")
    f = pl.kernel(
        _emb_body,
        out_type=jax.ShapeDtypeStruct((ROW_LEN, D_MOD, N_ROWS), jnp.float32),
        mesh=mesh,
        scratch_types=[
            pltpu.VMEM((D_VOC * _STRIDE + 5,), jnp.float32),
            pltpu.VMEM((_JB, _IB), jnp.int32),
            pltpu.VMEM((_JB, _IB), jnp.int32),
            pltpu.VMEM((_JB, D_MOD, _IB), jnp.float32),
            pltpu.VMEM((_JB, D_MOD, _IB), jnp.float32),
            pltpu.SemaphoreType.DMA,
            pltpu.SemaphoreType.DMA,
            pltpu.SemaphoreType.DMA,
            pltpu.SemaphoreType.DMA,
        ],
        compiler_params=pltpu.CompilerParams(needs_layout_passes=False),
    )
    return f(tokens_t, table_pad)


def kernel(tokens, weight):
    tokens_t = tokens.T.astype(jnp.int32)
    table_pad = jnp.pad(weight, ((0, 0), (0, 1))).reshape(-1)
    table_pad = jnp.pad(table_pad, (0, 5))
    out_phys = _emb_call(tokens_t, table_pad)
    return jnp.transpose(out_phys, (2, 0, 1))


# speedup vs baseline: 2.4020x; 1.0103x over previous
import jax
import jax.numpy as jnp
from jax import lax
from jax.experimental import pallas as pl
from jax.experimental.pallas import tpu as pltpu
from jax.experimental.pallas import tpu_sc as plsc

D_VOC = 131
D_MOD = 32
N_ROWS = 16384
ROW_LEN = 200
_STRIDE = 33

_info = plsc.get_sparse_core_info()
_NC, _NS = _info.num_cores, _info.num_subcores
_NW = _NC * _NS

_I_PER_W = N_ROWS // _NW
_IB = 128
_JB = 8
_N_IBLK = _I_PER_W // _IB
_N_JBLK = ROW_LEN // _JB


_N_CHUNKS = _N_IBLK * _N_JBLK


def _emb_body(tokens_hbm, table_hbm, out_hbm, table_v,
              tok_v0, tok_v1, buf_v0, buf_v1, sem0, sem1, tsem0, tsem1):
    wid = lax.axis_index("s") * _NC + lax.axis_index("c")
    i_base = wid * _I_PER_W

    pltpu.sync_copy(table_hbm, table_v)
    toks = [tok_v0, tok_v1]
    bufs = [buf_v0, buf_v1]
    sems = [sem0, sem1]
    tsems = [tsem0, tsem1]

    def tok_slice(q):
        blk = q // _N_JBLK
        jq = q % _N_JBLK
        return tokens_hbm.at[pl.ds(jq * _JB, _JB),
                             pl.ds(i_base + blk * _IB, _IB)]

    def prefetch(q, b):
        pltpu.async_copy(tok_slice(q), toks[b], tsems[b])

    def compute(q, b):
        blk = q // _N_JBLK
        jq = q % _N_JBLK
        i0 = i_base + blk * _IB
        j0 = jq * _JB
        tok_v, buf_v = toks[b], bufs[b]
        pltpu.make_async_copy(tok_slice(q), tok_v, tsems[b]).wait()

        @plsc.parallel_loop(0, _JB, unroll=2)
        def jcol(jl):
            for g in range(_IB // 16):
                tok16 = tok_v[jl, pl.ds(g * 16, 16)]
                tok33 = tok16 * _STRIDE
                for dh in range(0, D_MOD, 8):
                    vals = [plsc.load_gather(table_v, [tok33 + (dh + d)])
                            for d in range(8)]
                    for d in range(8):
                        buf_v[jl, dh + d, pl.ds(g * 16, 16)] = vals[d]
        pltpu.async_copy(
            buf_v, out_hbm.at[pl.ds(j0, _JB), :, pl.ds(i0, _IB)], sems[b])

        @pl.when(q + 2 < _N_CHUNKS)
        def _():
            prefetch(q + 2, b)

    def drain(b):
        pltpu.make_async_copy(
            bufs[b], out_hbm.at[pl.ds(0, _JB), :, pl.ds(i_base, _IB)],
            sems[b]).wait()

    prefetch(0, 0)
    prefetch(1, 1)

    def super_step(s, carry):
        for b in range(2):
            q = s * 2 + b

            @pl.when(s > 0)
            def _():
                drain(b)
            compute(q, b)
        return carry

    lax.fori_loop(0, _N_CHUNKS // 2, super_step, 0)
    drain(0)
    drain(1)


@jax.jit
def _emb_call(tokens_t, table_pad):
    mesh = plsc.VectorSubcoreMesh(core_axis_name="c", subcore_axis_name="s")
    f = pl.kernel(
        _emb_body,
        out_type=jax.ShapeDtypeStruct((ROW_LEN, D_MOD, N_ROWS), jnp.float32),
        mesh=mesh,
        scratch_types=[
            pltpu.VMEM((D_VOC * _STRIDE + 5,), jnp.float32),
            pltpu.VMEM((_JB, _IB), jnp.int32),
            pltpu.VMEM((_JB, _IB), jnp.int32),
            pltpu.VMEM((_JB, D_MOD, _IB), jnp.float32),
            pltpu.VMEM((_JB, D_MOD, _IB), jnp.float32),
            pltpu.SemaphoreType.DMA,
            pltpu.SemaphoreType.DMA,
            pltpu.SemaphoreType.DMA,
            pltpu.SemaphoreType.DMA,
        ],
        compiler_params=pltpu.CompilerParams(needs_layout_passes=False),
    )
    return f(tokens_t, table_pad)


def kernel(tokens, weight):
    tokens_t = tokens.T.astype(jnp.int32)
    table_pad = jnp.pad(weight, ((0, 0), (0, 1))).reshape(-1)
    table_pad = jnp.pad(table_pad, (0, 5))
    out_phys = _emb_call(tokens_t, table_pad)
    return jnp.transpose(out_phys, (2, 0, 1))
